# Initial kernel scaffold; baseline (speedup 1.0000x reference)
import jax
import jax.numpy as jnp
from jax.experimental import pallas as pl


def kernel(inputs, table):
    raise NotImplementedError("write your pallas kernel here")


# speedup vs baseline: 1.1029x; 1.1029x over previous
import functools

import jax
import jax.numpy as jnp
from jax import lax
from jax.experimental import pallas as pl
from jax.experimental.pallas import tpu as pltpu
from jax.experimental.pallas import tpu_sc as plsc

_VOCAB = 1000000
_EMBED = 32
_BATCH = 16384
_SEQ = 50
_B = _BATCH * _SEQ
_NC = 2
_NS = 16
_NW = _NC * _NS
_BPW = _B // _NW
_CHUNK = 1600
_NCHUNK = _BPW // _CHUNK


@functools.partial(
    pl.kernel,
    out_type=jax.ShapeDtypeStruct((_B, _EMBED), jnp.float32),
    mesh=plsc.VectorSubcoreMesh(core_axis_name="c", subcore_axis_name="s"),
    scratch_types=[
        pltpu.VMEM((_CHUNK,), jnp.int32),
        pltpu.VMEM((_CHUNK, _EMBED), jnp.float32),
        pltpu.SemaphoreType.DMA,
    ],
    compiler_params=pltpu.CompilerParams(use_tc_tiling_on_sc=False),
)
def _embed_lookup(idx_hbm, table_hbm, out_hbm, idx_v, rows_v, sem):
    wid = lax.axis_index("s") * _NC + lax.axis_index("c")
    base = wid * _BPW

    @pl.loop(0, _NCHUNK)
    def _chunk(i):
        off = base + i * _CHUNK
        pltpu.sync_copy(idx_hbm.at[pl.ds(off, _CHUNK)], idx_v)
        pltpu.async_copy(table_hbm.at[idx_v], rows_v, sem).wait()
        pltpu.sync_copy(rows_v, out_hbm.at[pl.ds(off, _CHUNK)])


def kernel(inputs, table):
    flat_idx = inputs.reshape(_B).astype(jnp.int32)
    out = _embed_lookup(flat_idx, table)
    return out.reshape(_BATCH, _SEQ, _EMBED)


# speedup vs baseline: 1.1136x; 1.0097x over previous
import functools

import jax
import jax.numpy as jnp
from jax import lax
from jax.experimental import pallas as pl
from jax.experimental.pallas import tpu as pltpu
from jax.experimental.pallas import tpu_sc as plsc

_VOCAB = 1000000
_EMBED = 32
_BATCH = 16384
_SEQ = 50
_B = _BATCH * _SEQ
_NC = 2
_NS = 16
_NW = _NC * _NS
_BPW = _B // _NW
_CHUNK = 800
_NCHUNK = _BPW // _CHUNK
_NBUF = 4


@functools.partial(
    pl.kernel,
    out_type=jax.ShapeDtypeStruct((_B, _EMBED), jnp.float32),
    mesh=plsc.VectorSubcoreMesh(core_axis_name="c", subcore_axis_name="s"),
    scratch_types=(
        [pltpu.VMEM((_CHUNK,), jnp.int32) for _ in range(_NBUF)]
        + [pltpu.VMEM((_NBUF, _CHUNK, _EMBED), jnp.float32),
           pltpu.SemaphoreType.DMA((_NBUF,)),
           pltpu.SemaphoreType.DMA((_NBUF,)),
           pltpu.SemaphoreType.DMA((_NBUF,))]
    ),
    compiler_params=pltpu.CompilerParams(use_tc_tiling_on_sc=False),
)
def _embed_lookup(idx_hbm, table_hbm, out_hbm, i0, i1, i2, i3, rows_v,
                  sem_i, sem_g, sem_s):
    idx_bufs = [i0, i1, i2, i3]
    wid = lax.axis_index("s") * _NC + lax.axis_index("c")
    base = wid * _BPW

    def idx_load(g, b):
        src = idx_hbm.at[pl.ds(base + g * _CHUNK, _CHUNK)]
        return pltpu.make_async_copy(src, idx_bufs[b], sem_i.at[b])

    def gather(g, b):
        return pltpu.make_async_copy(
            table_hbm.at[idx_bufs[b]], rows_v.at[b], sem_g.at[b])

    def store(g, b):
        dst = out_hbm.at[pl.ds(base + g * _CHUNK, _CHUNK)]
        return pltpu.make_async_copy(rows_v.at[b], dst, sem_s.at[b])

    for b in range(_NBUF):
        idx_load(b, b).start()
    idx_load(0, 0).wait()
    gather(0, 0).start()

    def steady(g, b, skip_store_wait, start_next, refill):
        b1 = (b + 1) % _NBUF
        if start_next:
            if not skip_store_wait:
                store(g + 1 - _NBUF, b1).wait()
            idx_load(g + 1, b1).wait()
            gather(g + 1, b1).start()
        gather(g, b).wait()
        store(g, b).start()
        if refill:
            idx_load(g + _NBUF, b).start()

    for g in range(_NBUF):
        steady(g, g, skip_store_wait=(g + 1 < _NBUF), start_next=True,
               refill=True)

    @pl.loop(_NBUF, _NCHUNK - _NBUF, step=_NBUF)
    def _ring(g0):
        for b in range(_NBUF):
            steady(g0 + b, b, skip_store_wait=False, start_next=True,
                   refill=True)

    for b in range(_NBUF):
        g = _NCHUNK - _NBUF + b
        steady(g, b, skip_store_wait=False,
               start_next=(b + 1 < _NBUF), refill=False)

    for b in range(_NBUF):
        store(_NCHUNK - _NBUF + b, b).wait()


def kernel(inputs, table):
    flat_idx = inputs.reshape(_B).astype(jnp.int32)
    out = _embed_lookup(flat_idx, table)
    return out.reshape(_BATCH, _SEQ, _EMBED)


# speedup vs baseline: 1.5436x; 1.3861x over previous
import functools

import jax
import jax.numpy as jnp
from jax import lax
from jax.experimental import pallas as pl
from jax.experimental.pallas import tpu as pltpu
from jax.experimental.pallas import tpu_sc as plsc

_VOCAB = 1000000
_EMBED = 32
_BATCH = 16384
_SEQ = 50
_B = _BATCH * _SEQ
_NW = 32
_BPW = _B // _NW
_CHUNK = 512
_NCH = _BPW // _CHUNK
_NBUF = 2
_EB = _EMBED // 8
_BB = _BATCH // 128
_GB = _CHUNK // 128


@functools.partial(
    pl.kernel,
    out_type=jax.ShapeDtypeStruct((_SEQ, _EB, _BB, 8, 128), jnp.float32),
    mesh=plsc.VectorSubcoreMesh(core_axis_name="c", subcore_axis_name="s"),
    scratch_types=(
        [pltpu.VMEM((_CHUNK,), jnp.int32) for _ in range(_NBUF)]
        + [pltpu.VMEM((_NBUF, _CHUNK, _EMBED), jnp.float32),
           pltpu.VMEM((_NBUF, _EB, _GB, 8, 128), jnp.float32),
           pltpu.SemaphoreType.DMA((_NBUF,)),
           pltpu.SemaphoreType.DMA((_NBUF,)),
           pltpu.SemaphoreType.DMA((_NBUF,))]
    ),
    compiler_params=pltpu.CompilerParams(
        use_tc_tiling_on_sc=False, needs_layout_passes=False),
)
def _embed_lookup(idx_hbm, table_hbm, out_hbm, i0, i1, gbufs, tbufs,
                  sem_i, sem_g, sem_s):
    idx_bufs = [i0, i1]
    wid = lax.axis_index("s") * 2 + lax.axis_index("c")
    base = wid * _BPW
    blk0 = wid * (_BPW // 128)

    def idx_load(c, b):
        src = idx_hbm.at[pl.ds(base + c * _CHUNK, _CHUNK)]
        return pltpu.make_async_copy(src, idx_bufs[b], sem_i.at[b])

    def gather(b):
        return pltpu.make_async_copy(
            table_hbm.at[idx_bufs[b]], gbufs.at[b], sem_g.at[b])

    def transpose(b):
        lanes = lax.iota(jnp.int32, 16)

        @pl.loop(0, _EB * _GB * 8)
        def _t(t):
            eb = t // (_GB * 8)
            g = (t // 8) % _GB
            e8 = t % 8
            col = jnp.full((16,), eb * 8 + e8, jnp.int32)
            for sub in range(8):
                row = lanes + (g * 128 + sub * 16)
                val = plsc.load_gather(gbufs.at[b], [row, col])
                tbufs[b, eb, g, e8, pl.ds(sub * 16, 16)] = val

    def stores(c, b):
        blk = blk0 + c * _GB
        s = blk // _BB
        bb = blk % _BB
        return [
            pltpu.make_async_copy(
                tbufs.at[b, eb], out_hbm.at[s, eb, pl.ds(bb, _GB)],
                sem_s.at[b])
            for eb in range(_EB)
        ]

    idx_load(0, 0).start()
    idx_load(1, 1).start()
    idx_load(0, 0).wait()
    gather(0).start()

    def step(c, b, do_refill, do_store_wait, do_next_gather):
        b1 = (b + 1) % _NBUF
        gather(b).wait()
        if do_refill:
            idx_load(c + _NBUF, b).start()
        if do_store_wait:
            for d in stores(c - _NBUF, b):
                d.wait()
        transpose(b)
        for d in stores(c, b):
            d.start()
        if do_next_gather:
            idx_load(c + 1, b1).wait()
            gather(b1).start()

    step(0, 0, True, False, True)
    step(1, 1, True, False, True)

    @pl.loop(2, _NCH - 2, step=_NBUF)
    def _main(c0):
        step(c0, 0, True, True, True)
        step(c0 + 1, 1, True, True, True)

    step(_NCH - 2, 0, False, True, True)
    step(_NCH - 1, 1, False, True, False)

    for c, b in ((_NCH - 2, 0), (_NCH - 1, 1)):
        for d in stores(c, b):
            d.wait()


def kernel(inputs, table):
    idx_sm = jnp.swapaxes(inputs, 0, 1).reshape(_B).astype(jnp.int32)
    out5 = _embed_lookup(idx_sm, table)
    return out5.transpose(2, 4, 0, 1, 3).reshape(_BATCH, _SEQ, _EMBED)


# speedup vs baseline: 1.6481x; 1.0677x over previous
import functools

import jax
import jax.numpy as jnp
from jax import lax
from jax.experimental import pallas as pl
from jax.experimental.pallas import tpu as pltpu
from jax.experimental.pallas import tpu_sc as plsc

_VOCAB = 1000000
_EMBED = 32
_BATCH = 16384
_SEQ = 50
_B = _BATCH * _SEQ
_NW = 32
_BPW = _B // _NW
_CHUNK = 512
_NCH = _BPW // _CHUNK
_NBUF = 2
_EB = _EMBED // 8
_BB = _BATCH // 128
_GB = _CHUNK // 128


@functools.partial(
    pl.kernel,
    out_type=jax.ShapeDtypeStruct((_SEQ, _EB, _BB, 8, 128), jnp.float32),
    mesh=plsc.VectorSubcoreMesh(core_axis_name="c", subcore_axis_name="s"),
    scratch_types=(
        [pltpu.VMEM((_CHUNK,), jnp.int32) for _ in range(_NBUF)]
        + [pltpu.VMEM((_NBUF, _CHUNK, _EMBED), jnp.float32),
           pltpu.VMEM((_NBUF, _EB, _GB, 8, 128), jnp.float32),
           pltpu.SemaphoreType.DMA((_NBUF,)),
           pltpu.SemaphoreType.DMA((_NBUF,)),
           pltpu.SemaphoreType.DMA((_NBUF,))]
    ),
    compiler_params=pltpu.CompilerParams(
        use_tc_tiling_on_sc=False, needs_layout_passes=False),
)
def _embed_lookup(idx_hbm, table_hbm, out_hbm, i0, i1, gbufs, tbufs,
                  sem_i, sem_g, sem_s):
    idx_bufs = [i0, i1]
    wid = lax.axis_index("s") * 2 + lax.axis_index("c")
    base = wid * _BPW
    blk0 = wid * (_BPW // 128)

    def idx_load(c, b):
        src = idx_hbm.at[pl.ds(base + c * _CHUNK, _CHUNK)]
        return pltpu.make_async_copy(src, idx_bufs[b], sem_i.at[b])

    def gather(b):
        return pltpu.make_async_copy(
            table_hbm.at[idx_bufs[b]], gbufs.at[b], sem_g.at[b])

    def transpose(b):
        lanes = lax.iota(jnp.int32, 16)

        @pl.loop(0, _GB * 8)
        def _t(t):
            g = t // 8
            e8 = t % 8
            base = g * 128
            for eb in range(_EB):
                col = jnp.full((16,), eb * 8, jnp.int32) + e8
                for sub in range(8):
                    row = lanes + base + (sub * 16)
                    val = plsc.load_gather(gbufs.at[b], [row, col])
                    tbufs[b, eb, g, e8, pl.ds(sub * 16, 16)] = val

    def stores(c, b):
        blk = blk0 + c * _GB
        s = blk // _BB
        bb = blk % _BB
        return [
            pltpu.make_async_copy(
                tbufs.at[b, eb], out_hbm.at[s, eb, pl.ds(bb, _GB)],
                sem_s.at[b])
            for eb in range(_EB)
        ]

    idx_load(0, 0).start()
    idx_load(1, 1).start()
    idx_load(0, 0).wait()
    gather(0).start()

    def step(c, b, do_refill, do_store_wait, do_next_gather):
        b1 = (b + 1) % _NBUF
        if do_next_gather:
            idx_load(c + 1, b1).wait()
            gather(b1).start()
        gather(b).wait()
        if do_refill:
            idx_load(c + _NBUF, b).start()
        if do_store_wait:
            for d in stores(c - _NBUF, b):
                d.wait()
        transpose(b)
        for d in stores(c, b):
            d.start()

    step(0, 0, True, False, True)
    step(1, 1, True, False, True)

    @pl.loop(2, _NCH - 2, step=_NBUF)
    def _main(c0):
        step(c0, 0, True, True, True)
        step(c0 + 1, 1, True, True, True)

    step(_NCH - 2, 0, False, True, True)
    step(_NCH - 1, 1, False, True, False)

    for c, b in ((_NCH - 2, 0), (_NCH - 1, 1)):
        for d in stores(c, b):
            d.wait()


def kernel(inputs, table):
    idx_sm = jnp.swapaxes(inputs, 0, 1).reshape(_B).astype(jnp.int32)
    out5 = _embed_lookup(idx_sm, table)
    return out5.transpose(2, 4, 0, 1, 3).reshape(_BATCH, _SEQ, _EMBED)
